# baseline (speedup 1.0000x reference)
import functools

import jax
import jax.numpy as jnp
from jax.experimental import pallas as pl
from jax.experimental.pallas import tpu as pltpu

_N = 16
_NCELL = _N * _N * _N
_L_LOC = 10
_L_DIR = 4
_SCALE = 3.0
_CAP = 128
_G = 8


def _encode(p, L):
    parts = [p]
    for j in range(L):
        f = float(2.0 ** j)
        parts.append(jnp.sin(f * p))
        parts.append(jnp.cos(f * p))
    return jnp.concatenate(parts, axis=1)


def _mlp_body(xd_ref, w1_ref, b1_ref, w2_ref, b2_ref, w3_ref, b3_ref,
              w4_ref, b4_ref, w5_ref, b5_ref, out_ref):
    for g in range(_G):
        xd = xd_ref[g]
        xp = xd[:, 0:3]
        dp = xd[:, 3:6]
        enc_x = _encode(xp, _L_LOC)
        enc_d = _encode(dp, _L_DIR)

        h = jnp.dot(enc_x, w1_ref[g], preferred_element_type=jnp.float32)
        h = jax.nn.relu(h + b1_ref[g])
        h = jnp.dot(h, w2_ref[g], preferred_element_type=jnp.float32)
        h = jax.nn.relu(h + b2_ref[g])
        sigma = h[:, 0:1]
        h = h[:, 1:33]
        h = jnp.dot(h, w3_ref[g], preferred_element_type=jnp.float32) + b3_ref[g]
        h = jnp.concatenate([h, enc_d], axis=1)
        h = jnp.dot(h, w4_ref[g], preferred_element_type=jnp.float32)
        h = jax.nn.relu(h + b4_ref[g])
        h = jnp.dot(h, w5_ref[g], preferred_element_type=jnp.float32)
        rgb = jax.nn.sigmoid(h + b5_ref[g])

        half = _SCALE / 2
        mask = ((jnp.abs(xp[:, 0:1]) < half)
                & (jnp.abs(xp[:, 1:2]) < half)
                & (jnp.abs(xp[:, 2:3]) < half))
        out = jnp.concatenate([rgb, sigma], axis=1)
        out_ref[g] = jnp.where(mask, out, 0.0)


@jax.jit
def kernel(x, d, weight1, bias1, weight2, bias2, weight3, bias3,
           weight4, bias4, weight5, bias5):
    B = x.shape[0]

    i = jnp.clip((x / (_SCALE / _N) + _N / 2).astype(jnp.int32), 0, _N - 1)
    cid = i[:, 0] * (_N * _N) + i[:, 1] * _N + i[:, 2]
    order = jnp.argsort(cid)
    cid_s = cid[order]
    start_s = jnp.searchsorted(cid_s, cid_s, side='left')
    slot_s = jnp.arange(B, dtype=jnp.int32) - start_s.astype(jnp.int32)
    flat = cid_s * _CAP + slot_s

    xd = jnp.concatenate([x, d], axis=1)
    xd_pad = jnp.zeros((_NCELL * _CAP, 6), jnp.float32).at[flat].set(xd[order])
    xd_pad = xd_pad.reshape(_NCELL, _CAP, 6)

    w1 = weight1.reshape(_NCELL, 63, 32)
    w2 = weight2.reshape(_NCELL, 32, 33)
    w3 = weight3.reshape(_NCELL, 32, 32)
    w4 = weight4.reshape(_NCELL, 59, 32)
    w5 = weight5.reshape(_NCELL, 32, 3)
    b1 = bias1.reshape(_NCELL, 1, 32)
    b2 = bias2.reshape(_NCELL, 1, 33)
    b3 = bias3.reshape(_NCELL, 1, 32)
    b4 = bias4.reshape(_NCELL, 1, 32)
    b5 = bias5.reshape(_NCELL, 1, 3)

    def cell_map(ci):
        return (ci, 0, 0)

    grid = (_NCELL // _G,)
    out_pad = pl.pallas_call(
        _mlp_body,
        grid=grid,
        in_specs=[
            pl.BlockSpec((_G, _CAP, 6), cell_map),
            pl.BlockSpec((_G, 63, 32), cell_map),
            pl.BlockSpec((_G, 1, 32), cell_map),
            pl.BlockSpec((_G, 32, 33), cell_map),
            pl.BlockSpec((_G, 1, 33), cell_map),
            pl.BlockSpec((_G, 32, 32), cell_map),
            pl.BlockSpec((_G, 1, 32), cell_map),
            pl.BlockSpec((_G, 59, 32), cell_map),
            pl.BlockSpec((_G, 1, 32), cell_map),
            pl.BlockSpec((_G, 32, 3), cell_map),
            pl.BlockSpec((_G, 1, 3), cell_map),
        ],
        out_specs=pl.BlockSpec((_G, _CAP, 4), cell_map),
        out_shape=jax.ShapeDtypeStruct((_NCELL, _CAP, 4), jnp.float32),
        compiler_params=pltpu.CompilerParams(
            dimension_semantics=("arbitrary",)),
    )(xd_pad, w1, b1, w2, b2, w3, b3, w4, b4, w5, b5)

    out_s = out_pad.reshape(_NCELL * _CAP, 4)[flat]
    out = jnp.zeros((B, 4), jnp.float32).at[order].set(out_s)
    return (out[:, 0:3], out[:, 3:4])


# speedup vs baseline: 3.9096x; 3.9096x over previous
import jax
import jax.numpy as jnp
from jax.experimental import pallas as pl
from jax.experimental.pallas import tpu as pltpu

_N = 16
_NCELL = _N * _N * _N
_L_LOC = 10
_L_DIR = 4
_SCALE = 3.0
_BLK = 128


def _encode_T(p, L):
    parts = [p]
    for j in range(L):
        a = (2.0 ** j) * p
        parts.append(jnp.sin(a))
        parts.append(jnp.cos(a))
    return jnp.concatenate(parts, axis=0)


def _seg_body(blk_ref, cell_ref, lo_ref, hi_ref,
              xd_ref, w1_ref, b1_ref, w2_ref, b2_ref, w3_ref, b3_ref,
              w4_ref, b4_ref, w5_ref, b5_ref, out_ref):
    s = pl.program_id(0)
    lo = lo_ref[s]
    hi = hi_ref[s]

    @pl.when(lo < hi)
    def _():
        xp = xd_ref[0:3, :]
        dp = xd_ref[3:6, :]
        enc_x = _encode_T(xp, _L_LOC)
        enc_d = _encode_T(dp, _L_DIR)

        h = jnp.dot(w1_ref[0], enc_x, preferred_element_type=jnp.float32)
        h = jax.nn.relu(h + b1_ref[0])
        h = jnp.dot(w2_ref[0], h, preferred_element_type=jnp.float32)
        h = jax.nn.relu(h + b2_ref[0])
        sigma = h[0:1, :]
        h = h[1:33, :]
        h = jnp.dot(w3_ref[0], h, preferred_element_type=jnp.float32) + b3_ref[0]
        h = jnp.concatenate([h, enc_d], axis=0)
        h = jnp.dot(w4_ref[0], h, preferred_element_type=jnp.float32)
        h = jax.nn.relu(h + b4_ref[0])
        h = jnp.dot(w5_ref[0], h, preferred_element_type=jnp.float32)
        rgb = jax.nn.sigmoid(h + b5_ref[0])

        half = _SCALE / 2
        box = ((jnp.abs(xp[0:1, :]) < half)
               & (jnp.abs(xp[1:2, :]) < half)
               & (jnp.abs(xp[2:3, :]) < half))
        new = jnp.concatenate([rgb, sigma], axis=0)
        new = jnp.where(box, new, 0.0)
        lane = jax.lax.broadcasted_iota(jnp.int32, (4, _BLK), 1)
        sel = (lane >= lo) & (lane < hi)
        out_ref[:, :] = jnp.where(sel, new, out_ref[:, :])


@jax.jit
def kernel(x, d, weight1, bias1, weight2, bias2, weight3, bias3,
           weight4, bias4, weight5, bias5):
    B = x.shape[0]
    nblk = B // _BLK
    smax = nblk + _NCELL

    i = jnp.clip((x / (_SCALE / _N) + _N / 2).astype(jnp.int32), 0, _N - 1)
    cid = i[:, 0] * (_N * _N) + i[:, 1] * _N + i[:, 2]
    order = jnp.argsort(cid)
    cid_s = cid[order]

    xd_T = jnp.concatenate([x, d], axis=1)[order].T

    pos = jnp.arange(B, dtype=jnp.int32)
    is_start = (pos % _BLK == 0) | jnp.concatenate(
        [jnp.ones((1,), bool), cid_s[1:] != cid_s[:-1]])
    starts = jnp.nonzero(is_start, size=smax, fill_value=B)[0].astype(jnp.int32)
    ends = jnp.concatenate([starts[1:], jnp.full((1,), B, jnp.int32)])
    seg_blk = jnp.minimum(starts // _BLK, nblk - 1)
    seg_lo = jnp.minimum(starts - seg_blk * _BLK, _BLK)
    seg_hi = jnp.clip(ends - seg_blk * _BLK, seg_lo, _BLK)
    seg_cell = cid_s[jnp.minimum(starts, B - 1)]

    w1 = jnp.swapaxes(weight1.reshape(_NCELL, 63, 32), 1, 2)
    w2 = jnp.swapaxes(weight2.reshape(_NCELL, 32, 33), 1, 2)
    w3 = jnp.swapaxes(weight3.reshape(_NCELL, 32, 32), 1, 2)
    w4 = jnp.swapaxes(weight4.reshape(_NCELL, 59, 32), 1, 2)
    w5 = jnp.swapaxes(weight5.reshape(_NCELL, 32, 3), 1, 2)
    b1 = bias1.reshape(_NCELL, 32)[:, :, None]
    b2 = bias2.reshape(_NCELL, 33)[:, :, None]
    b3 = bias3.reshape(_NCELL, 32)[:, :, None]
    b4 = bias4.reshape(_NCELL, 32)[:, :, None]
    b5 = bias5.reshape(_NCELL, 3)[:, :, None]

    def m_pts(s, sb, sc, sl, sh):
        return (0, sb[s])

    def m_cell(s, sb, sc, sl, sh):
        return (sc[s], 0, 0)

    grid_spec = pltpu.PrefetchScalarGridSpec(
        num_scalar_prefetch=4,
        grid=(smax,),
        in_specs=[
            pl.BlockSpec((6, _BLK), m_pts),
            pl.BlockSpec((1, 32, 63), m_cell),
            pl.BlockSpec((1, 32, 1), m_cell),
            pl.BlockSpec((1, 33, 32), m_cell),
            pl.BlockSpec((1, 33, 1), m_cell),
            pl.BlockSpec((1, 32, 32), m_cell),
            pl.BlockSpec((1, 32, 1), m_cell),
            pl.BlockSpec((1, 32, 59), m_cell),
            pl.BlockSpec((1, 32, 1), m_cell),
            pl.BlockSpec((1, 3, 32), m_cell),
            pl.BlockSpec((1, 3, 1), m_cell),
        ],
        out_specs=pl.BlockSpec((4, _BLK), m_pts),
    )
    out_T = pl.pallas_call(
        _seg_body,
        grid_spec=grid_spec,
        out_shape=jax.ShapeDtypeStruct((4, B), jnp.float32),
        compiler_params=pltpu.CompilerParams(
            dimension_semantics=("arbitrary",)),
    )(seg_blk, seg_cell, seg_lo, seg_hi,
      xd_T, w1, b1, w2, b2, w3, b3, w4, b4, w5, b5)

    out = jnp.zeros((B, 4), jnp.float32).at[order].set(out_T.T)
    return (out[:, 0:3], out[:, 3:4])
